# baseline (speedup 1.0000x reference)
import functools

import jax
import jax.numpy as jnp
from jax import lax
from jax.experimental import pallas as pl
from jax.experimental.pallas import tpu as pltpu
from jax.experimental.pallas import tpu_sc as plsc

NUF = 26
NL = 16
BATCH = 16384
NC, NS, L = 2, 16, 16
NW = NC * NS
BPW = BATCH // NW
NTILE = BPW // L
NCHUNK = 4
CHUNK = BPW // NCHUNK
TPT = 2


def _sc_forward(ufT, ids3, tableT, w_vb, b_vb):
    mesh = plsc.VectorSubcoreMesh(core_axis_name="c", subcore_axis_name="s")

    @functools.partial(
        pl.kernel,
        mesh=mesh,
        out_type=jax.ShapeDtypeStruct((BATCH,), jnp.float32),
        compiler_params=pltpu.CompilerParams(use_tc_tiling_on_sc=False),
        scratch_types=[
            pltpu.VMEM((NCHUNK, CHUNK), jnp.int32),
            pltpu.VMEM((NL, BPW), jnp.float32),
            pltpu.VMEM((NUF, BPW), jnp.float32),
            pltpu.VMEM((NL * NUF // 8, 128), jnp.float32),
            pltpu.VMEM((NL // 8, 128), jnp.float32),
            pltpu.VMEM((NL, BPW), jnp.float32),
            pltpu.VMEM((BPW,), jnp.float32),
            pltpu.SemaphoreType.DMA,
        ],
    )
    def run(ufT_hbm, ids_hbm, tblT_hbm, w_hbm, b_hbm, out_hbm,
            idx_v, itemT_v, uft_v, w_v, b_v, ut_v, out_v, gsem):
        wid = lax.axis_index("s") * NC + lax.axis_index("c")
        base = wid * BPW

        pltpu.sync_copy(ids_hbm.at[wid], idx_v)
        gathers = []
        for l in range(NL):
            row = tblT_hbm.at[l]
            for j in range(NCHUNK):
                gathers.append(pltpu.async_copy(
                    row.at[idx_v.at[j]],
                    itemT_v.at[l, pl.ds(j * CHUNK, CHUNK)], gsem))
        pltpu.sync_copy(ufT_hbm.at[:, pl.ds(base, BPW)], uft_v)
        pltpu.sync_copy(w_hbm, w_v)
        pltpu.sync_copy(b_hbm, b_v)

        def user_pass(g, carry):
            offs = [(g * TPT + t) * L for t in range(TPT)]
            accs = [[b_v[l // 8, pl.ds((l % 8) * L, L)] for l in range(NL)]
                    for _ in range(TPT)]
            for k in range(NUF):
                ufk = [uft_v[k, pl.ds(offs[t], L)] for t in range(TPT)]
                for l in range(NL):
                    e = l * NUF + k
                    wv = w_v[e // 8, pl.ds((e % 8) * L, L)]
                    for t in range(TPT):
                        accs[t][l] = accs[t][l] + ufk[t] * wv
            for t in range(TPT):
                for l in range(NL):
                    ut_v[l, pl.ds(offs[t], L)] = accs[t][l]
            return carry

        lax.fori_loop(0, NTILE // TPT, user_pass, 0)

        for g in gathers:
            g.wait()

        def dot_pass(t, carry):
            off = t * L
            dot = jnp.zeros((L,), jnp.float32)
            for l in range(NL):
                dot = dot + ut_v[l, pl.ds(off, L)] * itemT_v[l, pl.ds(off, L)]
            out_v[pl.ds(off, L)] = 1.0 / (1.0 + jnp.exp(-dot))
            return carry

        lax.fori_loop(0, NTILE, dot_pass, 0)

        pltpu.sync_copy(out_v, out_hbm.at[pl.ds(base, BPW)])

    return run(ufT, ids3, tableT, w_vb, b_vb)


def kernel(user_features, item_ids, item_table, W, b):
    ufT = user_features.T
    tableT = item_table.T
    ids3 = item_ids.astype(jnp.int32).reshape(NW, NCHUNK, CHUNK)
    w_vb = jnp.broadcast_to(W.reshape(NL * NUF, 1),
                            (NL * NUF, L)).reshape(NL * NUF // 8, 128)
    b_vb = jnp.broadcast_to(b.reshape(NL, 1), (NL, L)).reshape(NL // 8, 128)
    return _sc_forward(ufT, ids3, tableT, w_vb, b_vb)


# speedup vs baseline: 3.6391x; 3.6391x over previous
import functools

import jax
import jax.numpy as jnp
from jax import lax
from jax.experimental import pallas as pl
from jax.experimental.pallas import tpu as pltpu
from jax.experimental.pallas import tpu_sc as plsc

NUF = 26
NL = 16
BATCH = 16384
NC, NS, L = 2, 16, 16
NW = NC * NS
BPW = BATCH // NW
NTILE = BPW // L
NCHUNK = 4
CHUNK = BPW // NCHUNK
TPT = 2


def _sc_forward(ufT, ids3, trows, w_vb, b_vb):
    mesh = plsc.VectorSubcoreMesh(core_axis_name="c", subcore_axis_name="s")

    @functools.partial(
        pl.kernel,
        mesh=mesh,
        out_type=jax.ShapeDtypeStruct((BATCH,), jnp.float32),
        compiler_params=pltpu.CompilerParams(use_tc_tiling_on_sc=False),
        scratch_types=[
            pltpu.VMEM((NCHUNK, CHUNK), jnp.int32),
            pltpu.VMEM((NL, BPW), jnp.float32),
            pltpu.VMEM((NUF, BPW), jnp.float32),
            pltpu.VMEM((NL * NUF // 8, 128), jnp.float32),
            pltpu.VMEM((NL // 8, 128), jnp.float32),
            pltpu.VMEM((NL, BPW), jnp.float32),
            pltpu.VMEM((BPW,), jnp.float32),
            pltpu.SemaphoreType.DMA,
        ],
    )
    def run(ufT_hbm, ids_hbm, *rest):
        (t0, t1, t2, t3, t4, t5, t6, t7,
         t8, t9, t10, t11, t12, t13, t14, t15,
         w_hbm, b_hbm, out_hbm,
         idx_v, itemT_v, uft_v, w_v, b_v, ut_v, out_v, gsem) = rest
        trows_hbm = [t0, t1, t2, t3, t4, t5, t6, t7,
                     t8, t9, t10, t11, t12, t13, t14, t15]
        wid = lax.axis_index("s") * NC + lax.axis_index("c")
        base = wid * BPW

        pltpu.sync_copy(ids_hbm.at[wid], idx_v)
        gathers = []
        for l in range(NL):
            row = trows_hbm[l]
            for j in range(NCHUNK):
                gathers.append(pltpu.async_copy(
                    row.at[idx_v.at[j]],
                    itemT_v.at[l, pl.ds(j * CHUNK, CHUNK)], gsem))
        pltpu.sync_copy(ufT_hbm.at[:, pl.ds(base, BPW)], uft_v)
        pltpu.sync_copy(w_hbm, w_v)
        pltpu.sync_copy(b_hbm, b_v)

        def user_pass(g, carry):
            offs = [(g * TPT + t) * L for t in range(TPT)]
            accs = [[b_v[l // 8, pl.ds((l % 8) * L, L)] for l in range(NL)]
                    for _ in range(TPT)]
            for k in range(NUF):
                ufk = [uft_v[k, pl.ds(offs[t], L)] for t in range(TPT)]
                for l in range(NL):
                    e = l * NUF + k
                    wv = w_v[e // 8, pl.ds((e % 8) * L, L)]
                    for t in range(TPT):
                        accs[t][l] = accs[t][l] + ufk[t] * wv
            for t in range(TPT):
                for l in range(NL):
                    ut_v[l, pl.ds(offs[t], L)] = accs[t][l]
            return carry

        lax.fori_loop(0, NTILE // TPT, user_pass, 0)

        for g in gathers:
            g.wait()

        def dot_pass(t, carry):
            off = t * L
            dot = jnp.zeros((L,), jnp.float32)
            for l in range(NL):
                dot = dot + ut_v[l, pl.ds(off, L)] * itemT_v[l, pl.ds(off, L)]
            out_v[pl.ds(off, L)] = 1.0 / (1.0 + jnp.exp(-dot))
            return carry

        lax.fori_loop(0, NTILE, dot_pass, 0)

        pltpu.sync_copy(out_v, out_hbm.at[pl.ds(base, BPW)])

    return run(ufT, ids3, *trows, w_vb, b_vb)


def kernel(user_features, item_ids, item_table, W, b):
    ufT = user_features.T
    trows = [item_table[:, l] for l in range(NL)]
    ids3 = item_ids.astype(jnp.int32).reshape(NW, NCHUNK, CHUNK)
    w_vb = jnp.broadcast_to(W.reshape(NL * NUF, 1),
                            (NL * NUF, L)).reshape(NL * NUF // 8, 128)
    b_vb = jnp.broadcast_to(b.reshape(NL, 1), (NL, L)).reshape(NL // 8, 128)
    return _sc_forward(ufT, ids3, trows, w_vb, b_vb)


# speedup vs baseline: 14.0176x; 3.8519x over previous
import functools

import jax
import jax.numpy as jnp
from jax import lax
from jax.experimental import pallas as pl
from jax.experimental.pallas import tpu as pltpu
from jax.experimental.pallas import tpu_sc as plsc

NUF = 26
NL = 16
BATCH = 16384
NC, NS, L = 2, 16, 16
NW = NC * NS
BPW = BATCH // NW
NTILE = BPW // L
NCHUNK = 4
CHUNK = BPW // NCHUNK
TPT = 2


TBL_BC = 31744


def _tc_detile(tbl3):
    n = tbl3.shape[2]
    grid = ((n + TBL_BC - 1) // TBL_BC,)
    out_shapes = [jax.ShapeDtypeStruct((n,), jnp.float32) for _ in range(NL)]

    def body(in_ref, *out_refs):
        for j in range(NL):
            out_refs[j][...] = in_ref[j // 8, j % 8, :]

    return pl.pallas_call(
        body,
        grid=grid,
        in_specs=[pl.BlockSpec((2, 8, TBL_BC), lambda i: (0, 0, i))],
        out_specs=[pl.BlockSpec((TBL_BC,), lambda i: (i,))
                   for _ in range(NL)],
        out_shape=out_shapes,
    )(tbl3)


def _sc_forward(ufT, ids3, trows, w_vb, b_vb):
    mesh = plsc.VectorSubcoreMesh(core_axis_name="c", subcore_axis_name="s")

    @functools.partial(
        pl.kernel,
        mesh=mesh,
        out_type=jax.ShapeDtypeStruct((BATCH,), jnp.float32),
        compiler_params=pltpu.CompilerParams(use_tc_tiling_on_sc=False),
        scratch_types=[
            pltpu.VMEM((NCHUNK, CHUNK), jnp.int32),
            pltpu.VMEM((NL, BPW), jnp.float32),
            pltpu.VMEM((NUF, BPW), jnp.float32),
            pltpu.VMEM((NL * NUF // 8, 128), jnp.float32),
            pltpu.VMEM((NL // 8, 128), jnp.float32),
            pltpu.VMEM((NL, BPW), jnp.float32),
            pltpu.VMEM((BPW,), jnp.float32),
            pltpu.SemaphoreType.DMA,
        ],
    )
    def run(ufT_hbm, ids_hbm, *rest):
        (t0, t1, t2, t3, t4, t5, t6, t7,
         t8, t9, t10, t11, t12, t13, t14, t15,
         w_hbm, b_hbm, out_hbm,
         idx_v, itemT_v, uft_v, w_v, b_v, ut_v, out_v, gsem) = rest
        trows_hbm = [t0, t1, t2, t3, t4, t5, t6, t7,
                     t8, t9, t10, t11, t12, t13, t14, t15]
        wid = lax.axis_index("s") * NC + lax.axis_index("c")
        base = wid * BPW

        pltpu.sync_copy(ids_hbm.at[wid], idx_v)
        gathers = []
        for l in range(NL):
            row = trows_hbm[l]
            for j in range(NCHUNK):
                gathers.append(pltpu.async_copy(
                    row.at[idx_v.at[j]],
                    itemT_v.at[l, pl.ds(j * CHUNK, CHUNK)], gsem))
        pltpu.sync_copy(ufT_hbm.at[:, pl.ds(base, BPW)], uft_v)
        pltpu.sync_copy(w_hbm, w_v)
        pltpu.sync_copy(b_hbm, b_v)

        def user_pass(g, carry):
            offs = [(g * TPT + t) * L for t in range(TPT)]
            accs = [[b_v[l // 8, pl.ds((l % 8) * L, L)] for l in range(NL)]
                    for _ in range(TPT)]
            for k in range(NUF):
                ufk = [uft_v[k, pl.ds(offs[t], L)] for t in range(TPT)]
                for l in range(NL):
                    e = l * NUF + k
                    wv = w_v[e // 8, pl.ds((e % 8) * L, L)]
                    for t in range(TPT):
                        accs[t][l] = accs[t][l] + ufk[t] * wv
            for t in range(TPT):
                for l in range(NL):
                    ut_v[l, pl.ds(offs[t], L)] = accs[t][l]
            return carry

        lax.fori_loop(0, NTILE // TPT, user_pass, 0)

        for g in gathers:
            g.wait()

        def dot_pass(t, carry):
            off = t * L
            dot = jnp.zeros((L,), jnp.float32)
            for l in range(NL):
                dot = dot + ut_v[l, pl.ds(off, L)] * itemT_v[l, pl.ds(off, L)]
            out_v[pl.ds(off, L)] = 1.0 / (1.0 + jnp.exp(-dot))
            return carry

        lax.fori_loop(0, NTILE, dot_pass, 0)

        pltpu.sync_copy(out_v, out_hbm.at[pl.ds(base, BPW)])

    return run(ufT, ids3, *trows, w_vb, b_vb)


def kernel(user_features, item_ids, item_table, W, b):
    ufT = user_features.T
    trows = _tc_detile(item_table.T.reshape(2, 8, 1000000))
    ids3 = item_ids.astype(jnp.int32).reshape(NW, NCHUNK, CHUNK)
    w_vb = jnp.broadcast_to(W.reshape(NL * NUF, 1),
                            (NL * NUF, L)).reshape(NL * NUF // 8, 128)
    b_vb = jnp.broadcast_to(b.reshape(NL, 1), (NL, L)).reshape(NL // 8, 128)
    return _sc_forward(ufT, ids3, trows, w_vb, b_vb)


# speedup vs baseline: 15.1873x; 1.0834x over previous
import functools

import jax
import jax.numpy as jnp
from jax import lax
from jax.experimental import pallas as pl
from jax.experimental.pallas import tpu as pltpu
from jax.experimental.pallas import tpu_sc as plsc

NUF = 26
NL = 16
BATCH = 16384
NC, NS, L = 2, 16, 16
NW = NC * NS
BPW = BATCH // NW
NTILE = BPW // L
NCHUNK = 4
CHUNK = BPW // NCHUNK
TPT = 2


TBL_BC = 63488


def _tc_detile(tbl3):
    n = tbl3.shape[2]
    grid = ((n + TBL_BC - 1) // TBL_BC,)
    out_shapes = [jax.ShapeDtypeStruct((n,), jnp.float32) for _ in range(NL)]

    def body(in_ref, *out_refs):
        for j in range(NL):
            out_refs[j][...] = in_ref[j // 8, j % 8, :]

    return pl.pallas_call(
        body,
        grid=grid,
        in_specs=[pl.BlockSpec((2, 8, TBL_BC), lambda i: (0, 0, i))],
        out_specs=[pl.BlockSpec((TBL_BC,), lambda i: (i,))
                   for _ in range(NL)],
        out_shape=out_shapes,
    )(tbl3)


def _sc_forward(ufT, ids3, trows, w_vb, b_vb):
    mesh = plsc.VectorSubcoreMesh(core_axis_name="c", subcore_axis_name="s")

    @functools.partial(
        pl.kernel,
        mesh=mesh,
        out_type=jax.ShapeDtypeStruct((BATCH,), jnp.float32),
        compiler_params=pltpu.CompilerParams(use_tc_tiling_on_sc=False),
        scratch_types=[
            pltpu.VMEM((NCHUNK, CHUNK), jnp.int32),
            pltpu.VMEM((NL, BPW), jnp.float32),
            pltpu.VMEM((NUF, BPW), jnp.float32),
            pltpu.VMEM((NL * NUF // 8, 128), jnp.float32),
            pltpu.VMEM((NL // 8, 128), jnp.float32),
            pltpu.VMEM((NL, BPW), jnp.float32),
            pltpu.VMEM((BPW,), jnp.float32),
            pltpu.SemaphoreType.DMA,
        ],
    )
    def run(ufT_hbm, ids_hbm, *rest):
        (t0, t1, t2, t3, t4, t5, t6, t7,
         t8, t9, t10, t11, t12, t13, t14, t15,
         w_hbm, b_hbm, out_hbm,
         idx_v, itemT_v, uft_v, w_v, b_v, ut_v, out_v, gsem) = rest
        trows_hbm = [t0, t1, t2, t3, t4, t5, t6, t7,
                     t8, t9, t10, t11, t12, t13, t14, t15]
        wid = lax.axis_index("s") * NC + lax.axis_index("c")
        base = wid * BPW

        pltpu.sync_copy(ids_hbm.at[wid], idx_v)
        gathers = []
        for l in range(NL):
            row = trows_hbm[l]
            for j in range(NCHUNK):
                gathers.append(pltpu.async_copy(
                    row.at[idx_v.at[j]],
                    itemT_v.at[l, pl.ds(j * CHUNK, CHUNK)], gsem))
        pltpu.sync_copy(ufT_hbm.at[:, pl.ds(base, BPW)], uft_v)
        pltpu.sync_copy(w_hbm, w_v)
        pltpu.sync_copy(b_hbm, b_v)

        def user_pass(g, carry):
            offs = [(g * TPT + t) * L for t in range(TPT)]
            accs = [[b_v[l // 8, pl.ds((l % 8) * L, L)] for l in range(NL)]
                    for _ in range(TPT)]
            for k in range(NUF):
                ufk = [uft_v[k, pl.ds(offs[t], L)] for t in range(TPT)]
                for l in range(NL):
                    e = l * NUF + k
                    wv = w_v[e // 8, pl.ds((e % 8) * L, L)]
                    for t in range(TPT):
                        accs[t][l] = accs[t][l] + ufk[t] * wv
            for t in range(TPT):
                for l in range(NL):
                    ut_v[l, pl.ds(offs[t], L)] = accs[t][l]
            return carry

        lax.fori_loop(0, NTILE // TPT, user_pass, 0)

        for g in gathers:
            g.wait()

        def dot_pass(t, carry):
            off = t * L
            dot = jnp.zeros((L,), jnp.float32)
            for l in range(NL):
                dot = dot + ut_v[l, pl.ds(off, L)] * itemT_v[l, pl.ds(off, L)]
            out_v[pl.ds(off, L)] = 1.0 / (1.0 + jnp.exp(-dot))
            return carry

        lax.fori_loop(0, NTILE, dot_pass, 0)

        pltpu.sync_copy(out_v, out_hbm.at[pl.ds(base, BPW)])

    return run(ufT, ids3, *trows, w_vb, b_vb)


def kernel(user_features, item_ids, item_table, W, b):
    ufT = user_features.T
    trows = _tc_detile(item_table.T.reshape(2, 8, 1000000))
    ids3 = item_ids.astype(jnp.int32).reshape(NW, NCHUNK, CHUNK)
    w_vb = jnp.broadcast_to(W.reshape(NL * NUF, 1),
                            (NL * NUF, L)).reshape(NL * NUF // 8, 128)
    b_vb = jnp.broadcast_to(b.reshape(NL, 1), (NL, L)).reshape(NL // 8, 128)
    return _sc_forward(ufT, ids3, trows, w_vb, b_vb)
